# Initial kernel scaffold; baseline (speedup 1.0000x reference)
import jax
import jax.numpy as jnp
from jax.experimental import pallas as pl


def kernel(x, W, b):
    raise NotImplementedError("write your pallas kernel here")


# speedup vs baseline: 9.0026x; 9.0026x over previous
import jax
import jax.numpy as jnp
from jax.experimental import pallas as pl

K_NN = 20


def _knn_feat_kernel(xt_blk_ref, x_all_ref, w1_ref, wd_ref, bias_ref, out_ref):
    xt_blk = xt_blk_ref[...]
    x_all = x_all_ref[...]
    R = xt_blk.shape[0]
    N = x_all.shape[1]

    r2 = jnp.sum(xt_blk * xt_blk, axis=1, keepdims=True)
    c2 = jnp.sum(x_all * x_all, axis=0, keepdims=True)
    d = 2.0 * jnp.dot(xt_blk, x_all, preferred_element_type=jnp.float32)
    d = d - r2 - c2

    iota = jax.lax.broadcasted_iota(jnp.int32, (R, N), 1)
    neg_inf = jnp.float32(-jnp.inf)

    def body(_, dcur):
        m = jnp.max(dcur, axis=1, keepdims=True)
        cand = jnp.where(dcur == m, iota, jnp.int32(N))
        a = jnp.min(cand, axis=1, keepdims=True)
        return jnp.where(iota == a, neg_inf, dcur)

    d_final = jax.lax.fori_loop(0, K_NN, body, d)
    mask = (d_final == neg_inf).astype(jnp.float32)

    g = jax.lax.dot_general(mask, x_all, (((1,), (1,)), ((), ())),
                            preferred_element_type=jnp.float32)
    g = g * jnp.float32(1.0 / K_NN)

    o = jax.lax.dot_general(g, w1_ref[...], (((1,), (1,)), ((), ())),
                            preferred_element_type=jnp.float32)
    o = o + jax.lax.dot_general(xt_blk, wd_ref[...], (((1,), (1,)), ((), ())),
                                preferred_element_type=jnp.float32)
    out_ref[...] = o + bias_ref[...]


def kernel(x, W, b):
    B, C, N = x.shape
    O = W.shape[0]
    Wm = W[:, :, 0, 0]
    w1 = Wm[:, :C]
    wd = Wm[:, C:] - w1
    xt = jnp.transpose(x, (0, 2, 1))
    bias = b[None, :]

    R = min(256, N)
    grid = (B, N // R)

    out = pl.pallas_call(
        _knn_feat_kernel,
        grid=grid,
        in_specs=[
            pl.BlockSpec((None, R, C), lambda bb, i: (bb, i, 0)),
            pl.BlockSpec((None, C, N), lambda bb, i: (bb, 0, 0)),
            pl.BlockSpec((O, C), lambda bb, i: (0, 0)),
            pl.BlockSpec((O, C), lambda bb, i: (0, 0)),
            pl.BlockSpec((1, O), lambda bb, i: (0, 0)),
        ],
        out_specs=pl.BlockSpec((None, R, O), lambda bb, i: (bb, i, 0)),
        out_shape=jax.ShapeDtypeStruct((B, N, O), jnp.float32),
    )(xt, x, w1, wd, bias)

    return jnp.transpose(out, (0, 2, 1))
